# baseline (speedup 1.0000x reference)
import functools

import jax
import jax.numpy as jnp
from jax import lax
from jax.experimental import pallas as pl
from jax.experimental.pallas import tpu as pltpu
from jax.experimental.pallas import tpu_sc as plsc

H = 128
_NC = 2
_NS = 16
_NW = _NC * _NS
_K = 80


def _full(shape):
    return pl.BlockSpec(shape, lambda i: (0,) * len(shape))


def _rows(bsize, ncols):
    return pl.BlockSpec((bsize, ncols), lambda i: (i, 0))


def _softmax_scores(pts, pos):
    cross = lax.dot_general(pts, pos, (((1,), (1,)), ((), ())))
    pn = jnp.sum(pos * pos, axis=1)[None, :]
    logits = 2.0 * cross - pn
    m = jnp.max(logits, axis=1, keepdims=True)
    e = jnp.exp(logits - m)
    return e / jnp.sum(e, axis=1, keepdims=True)


def _enc_latents_kernel(xs_ref, pos_ref, w0_ref, b0_ref, w1_ref, b1_ref,
                        w2_ref, b2_ref, out_ref):
    i = pl.program_id(0)
    xs = xs_ref[...]
    h = jnp.maximum(jnp.dot(xs, w0_ref[...]) + b0_ref[...], 0.0)
    h = jnp.maximum(jnp.dot(h, w1_ref[...]) + b1_ref[...], 0.0)
    emb = jnp.dot(h, w2_ref[...]) + b2_ref[...]
    scores = _softmax_scores(xs, pos_ref[...])
    contrib = lax.dot_general(scores, emb, (((0,), (0,)), ((), ())))

    @pl.when(i == 0)
    def _():
        out_ref[...] = jnp.zeros_like(out_ref)

    out_ref[...] += contrib


def _edge_enc_kernel(e0_ref, w0_ref, b0_ref, w1_ref, b1_ref,
                     w2_ref, b2_ref, out_ref):
    e0 = e0_ref[...]
    h = jnp.maximum(e0 * w0_ref[...] + b0_ref[...], 0.0)
    h = jnp.maximum(jnp.dot(h, w1_ref[...]) + b1_ref[...], 0.0)
    out_ref[...] = jnp.dot(h, w2_ref[...]) + b2_ref[...]


def _proj_kernel(n_ref, wr_ref, ws_ref, a_ref, b_ref):
    n = n_ref[...]
    a_ref[...] = jnp.dot(n, wr_ref[...]).astype(a_ref.dtype)
    b_ref[...] = jnp.dot(n, ws_ref[...]).astype(b_ref.dtype)


def _edge_mlp_kernel(e_ref, g_ref, w0_ref, b0_ref, w1_ref, b1_ref,
                     w2_ref, b2_ref, out_ref):
    h = jnp.dot(e_ref[...], w0_ref[...]) + g_ref[...] + b0_ref[...]
    h = jnp.maximum(h, 0.0)
    h = jnp.maximum(jnp.dot(h, w1_ref[...]) + b1_ref[...], 0.0)
    out_ref[...] = jnp.dot(h, w2_ref[...]) + b2_ref[...]


def _node_kernel(n_ref, p0_ref, p1_ref, p2_ref, p3_ref, w0a_ref, w0b_ref,
                 b0_ref, w1_ref, b1_ref, w2_ref, b2_ref, wr_ref, ws_ref,
                 out_ref, a_ref, b_ref):
    n = n_ref[...]
    inbox = (p0_ref[...] + p1_ref[...]) + (p2_ref[...] + p3_ref[...])
    h = jnp.maximum(jnp.dot(n, w0a_ref[...]) + jnp.dot(inbox, w0b_ref[...])
                    + b0_ref[...], 0.0)
    h = jnp.maximum(jnp.dot(h, w1_ref[...]) + b1_ref[...], 0.0)
    nn = n + jnp.dot(h, w2_ref[...]) + b2_ref[...]
    out_ref[...] = nn
    a_ref[...] = jnp.dot(nn, wr_ref[...])
    b_ref[...] = jnp.dot(nn, ws_ref[...])


def _decode_kernel(qs_ref, pos_ref, lat_ref, w0z_ref, w0q_ref, b0_ref,
                   w1_ref, b1_ref, w2_ref, b2_ref, out_ref):
    qs = qs_ref[...]
    scores = _softmax_scores(qs, pos_ref[...])
    z = jnp.dot(scores, lat_ref[...])
    h = jnp.maximum(jnp.dot(z, w0z_ref[...]) + jnp.dot(qs, w0q_ref[...])
                    + b0_ref[...], 0.0)
    h = jnp.maximum(jnp.dot(h, w1_ref[...]) + b1_ref[...], 0.0)
    out_ref[...] = jnp.dot(h, w2_ref[...]) + b2_ref[...]


def _pick_k(per_w):
    for k in range(128, 7, -8):
        if per_w % k == 0:
            return k
    return 8


def _ring(per_w):
    k = _pick_k(per_w)
    n_chunks = per_w // k
    nb = min(6, n_chunks)
    return k, n_chunks, nb


def _run_ring(n_chunks, nb, fire0, steps):
    n_groups = n_chunks // nb - 1
    for b in range(nb):
        fire0(b, b)

    def group(j0, carry):
        for b in range(nb):
            c = nb * j0 + b
            steps(c, b)
            fire0(c + nb, b)
        return carry

    lax.fori_loop(0, n_groups, group, 0)
    for c in range(nb * n_groups, n_chunks):
        b = c % nb
        steps(c, b)
        nxt = c + nb
        if nxt < n_chunks:
            fire0(nxt, b)


def _sc_edge_len(E, NP):
    per_w = E // _NW
    mesh = plsc.VectorSubcoreMesh(core_axis_name="c", subcore_axis_name="s",
                                  num_cores=_NC, num_subcores=_NS)

    @functools.partial(
        pl.kernel,
        out_type=jax.ShapeDtypeStruct((E,), jnp.float32),
        mesh=mesh,
        compiler_params=pltpu.CompilerParams(needs_layout_passes=False),
        scratch_types=[
            pltpu.VMEM((NP,), jnp.float32),
            pltpu.VMEM((NP,), jnp.float32),
            pltpu.VMEM((NP,), jnp.float32),
            pltpu.VMEM((per_w,), jnp.int32),
            pltpu.VMEM((per_w,), jnp.int32),
            pltpu.VMEM((per_w,), jnp.float32),
        ],
    )
    def elen(px_hbm, py_hbm, pz_hbm, ir_hbm, is_hbm, out_hbm,
             px_v, py_v, pz_v, ir_v, is_v, e_v):
        wid = lax.axis_index("s") * _NC + lax.axis_index("c")
        base = wid * per_w
        pltpu.sync_copy(px_hbm, px_v)
        pltpu.sync_copy(py_hbm, py_v)
        pltpu.sync_copy(pz_hbm, pz_v)
        pltpu.sync_copy(ir_hbm.at[pl.ds(base, per_w)], ir_v)
        pltpu.sync_copy(is_hbm.at[pl.ds(base, per_w)], is_v)

        def body(t, carry):
            ir = ir_v[pl.ds(t * 16, 16)]
            js = is_v[pl.ds(t * 16, 16)]
            dx = plsc.load_gather(px_v, [ir]) - plsc.load_gather(px_v, [js])
            dy = plsc.load_gather(py_v, [ir]) - plsc.load_gather(py_v, [js])
            dz = plsc.load_gather(pz_v, [ir]) - plsc.load_gather(pz_v, [js])
            e_v[pl.ds(t * 16, 16)] = dx * dx + dy * dy + dz * dz
            return carry

        lax.fori_loop(0, per_w // 16, body, 0)
        pltpu.sync_copy(e_v, out_hbm.at[pl.ds(base, per_w)])

    return elen


def _sc_gather2(E, D):
    per_w = E // _NW
    K, n_chunks, NB = _ring(per_w)
    n_groups = n_chunks // NB - 1
    f32 = jnp.float32
    mesh = plsc.VectorSubcoreMesh(core_axis_name="c", subcore_axis_name="s",
                                  num_cores=_NC, num_subcores=_NS)

    @functools.partial(
        pl.kernel,
        out_type=jax.ShapeDtypeStruct((E, D), f32),
        mesh=mesh,
        scratch_types=(
            [pltpu.VMEM((per_w,), jnp.int32)] * 2
            + [pltpu.VMEM((K, D), f32)] * NB
            + [pltpu.SemaphoreType.DMA] * (3 * NB)
        ),
    )
    def gather2(t1_hbm, t2_hbm, i1_hbm, i2_hbm, o_hbm, *scr):
        i1_v, i2_v = scr[0], scr[1]
        r1 = scr[2:2 + NB]
        sg = scr[2 + NB:2 + 2 * NB]
        sa = scr[2 + 2 * NB:2 + 3 * NB]
        sw = scr[2 + 3 * NB:2 + 4 * NB]
        wid = lax.axis_index("s") * _NC + lax.axis_index("c")
        base = wid * per_w
        pltpu.sync_copy(i1_hbm.at[pl.ds(base, per_w)], i1_v)
        pltpu.sync_copy(i2_hbm.at[pl.ds(base, per_w)], i2_v)

        def fire_g(c, b):
            pltpu.async_copy(t1_hbm.at[i1_v.at[pl.ds(c * K, K)]], r1[b], sg[b])

        def wait_g(b):
            pltpu.make_async_copy(t1_hbm.at[pl.ds(0, K)], r1[b], sg[b]).wait()

        def fire_a(c, b):
            pltpu.async_copy(t2_hbm.at[i2_v.at[pl.ds(c * K, K)]], r1[b],
                             sa[b], add=True)

        def wait_a(b):
            pltpu.make_async_copy(t2_hbm.at[pl.ds(0, K)], r1[b], sa[b]).wait()

        def fire_w(c, b):
            pltpu.async_copy(r1[b], o_hbm.at[pl.ds(base + c * K, K)], sw[b])

        def wait_w(c, b):
            pltpu.make_async_copy(r1[b], o_hbm.at[pl.ds(base + c * K, K)],
                                  sw[b]).wait()

        def steps(c, b):
            wait_g(b)
            fire_a(c, b)
            wait_a(b)
            fire_w(c, b)
            wait_w(c, b)

        _run_ring(n_chunks, NB, fire_g, steps)

    return gather2


_KZ = 80


def _sc_scatter_add(E, D, n_rows):
    per_w = E // _NW
    K = min(_pick_k(per_w), 80)
    n_chunks = per_w // K
    NB = min(4, n_chunks)
    rows_per_tile = n_rows // _NS
    zchunks = rows_per_tile // _KZ
    f32 = jnp.float32
    mesh = plsc.VectorSubcoreMesh(core_axis_name="c", subcore_axis_name="s",
                                  num_cores=_NC, num_subcores=_NS)

    @functools.partial(
        pl.kernel,
        out_type=jax.ShapeDtypeStruct((_NC, n_rows, D), f32),
        mesh=mesh,
        scratch_types=(
            [pltpu.VMEM((K,), jnp.int32)] * NB
            + [pltpu.VMEM((K, D), f32)] * NB
            + [pltpu.SemaphoreType.DMA] * (2 * NB)
            + [pltpu.VMEM_SHARED((n_rows, D), f32)]
        ),
    )
    def scatter(vals_hbm, idx_hbm, zeros_hbm, out_hbm, *scr):
        ib = scr[0:NB]
        r = scr[NB:2 * NB]
        sl = scr[2 * NB:3 * NB]
        sa = scr[3 * NB:4 * NB]
        acc_sh = scr[4 * NB]
        cid = lax.axis_index("c")
        sid = lax.axis_index("s")
        wid = sid * _NC + cid
        base = wid * per_w
        row0 = sid * rows_per_tile

        pltpu.sync_copy(zeros_hbm, r[0].at[pl.ds(0, _KZ)])

        def zbody(j, carry):
            pltpu.sync_copy(r[0].at[pl.ds(0, _KZ)],
                            acc_sh.at[pl.ds(row0 + j * _KZ, _KZ)])
            return carry

        def fire_l(c, b):
            off = base + c * K
            pltpu.async_copy(idx_hbm.at[pl.ds(off, K)], ib[b], sl[b])
            pltpu.async_copy(vals_hbm.at[pl.ds(off, K)], r[b], sl[b])

        def wait_l(b):
            pltpu.make_async_copy(idx_hbm.at[pl.ds(0, K)], ib[b], sl[b]).wait()
            pltpu.make_async_copy(vals_hbm.at[pl.ds(base, K)], r[b],
                                  sl[b]).wait()

        def fire_s(c, b):
            pltpu.async_copy(r[b], acc_sh.at[ib[b]], sa[b], add=True)

        def wait_s(c, b):
            pltpu.make_async_copy(r[b], acc_sh.at[ib[b]], sa[b]).wait()

        lax.fori_loop(0, zchunks, zbody, 0)
        plsc.subcore_barrier()

        def steps(c, b):
            wait_l(b)
            fire_s(c, b)
            wait_s(c, b)

        _run_ring(n_chunks, NB, fire_l, steps)
        plsc.subcore_barrier()

        def dbody(j, carry):
            r0 = row0 + j * _KZ
            pltpu.sync_copy(acc_sh.at[pl.ds(r0, _KZ)],
                            r[0].at[pl.ds(0, _KZ)])
            pltpu.sync_copy(r[0].at[pl.ds(0, _KZ)],
                            out_hbm.at[cid, pl.ds(r0, _KZ)])
            return carry

        lax.fori_loop(0, zchunks, dbody, 0)

    return scatter


def kernel(x, s, q, pos, senders, receivers, params):
    f32 = jnp.float32
    N = pos.shape[0]
    E = senders.shape[0]
    NX = x.shape[1]
    NP = ((N + _NS * _K - 1) // (_NS * _K)) * (_NS * _K)
    BX = 256
    BE = 2560
    BN = 2560

    senders = senders.astype(jnp.int32)
    receivers = receivers.astype(jnp.int32)

    pos8 = jnp.zeros((NP, 8), f32).at[:N, :3].set(pos).at[N:, 0].set(1e4)
    xs8 = jnp.zeros((NX, 8), f32).at[:, :3].set(x[0]).at[:, 3:6].set(s[0])
    q8 = jnp.zeros((NX, 8), f32).at[:, :3].set(q[0])
    posx = jnp.zeros((NP,), f32).at[:N].set(pos[:, 0])
    posy = jnp.zeros((NP,), f32).at[:N].set(pos[:, 1])
    posz = jnp.zeros((NP,), f32).at[:N].set(pos[:, 2])

    enc = params["encoder"]
    w0e = jnp.zeros((8, H), f32).at[:6].set(enc["W0"])
    latents = pl.pallas_call(
        _enc_latents_kernel,
        grid=(NX // BX,),
        in_specs=[_rows(BX, 8), _full((NP, 8)), _full((8, H)), _full((1, H)),
                  _full((H, H)), _full((1, H)), _full((H, H)), _full((1, H))],
        out_specs=_full((NP, H)),
        out_shape=jax.ShapeDtypeStruct((NP, H), f32),
    )(xs8, pos8, w0e, enc["b0"][None], enc["W1"], enc["b1"][None],
      enc["W2"], enc["b2"][None])

    e0 = _sc_edge_len(E, NP)(posx, posy, posz, receivers, senders)

    q = _NW * _K
    e_a = ((E // q + 1) // 2) * q
    slabs = [(0, e_a), (e_a, E)]

    ee = params["edge_enc"]
    edges_s = []
    for lo, hi in slabs:
        edges_s.append(pl.pallas_call(
            _edge_enc_kernel,
            grid=((hi - lo) // BE,),
            in_specs=[_rows(BE, 1), _full((1, H)), _full((1, H)),
                      _full((H, H)), _full((1, H)), _full((H, H)),
                      _full((1, H))],
            out_specs=_rows(BE, H),
            out_shape=jax.ShapeDtypeStruct((hi - lo, H), f32),
        )(e0[lo:hi, None], ee["W0"], ee["b0"][None], ee["W1"], ee["b1"][None],
          ee["W2"], ee["b2"][None]))

    recv_s = [receivers[lo:hi] for lo, hi in slabs]
    send_s = [senders[lo:hi] for lo, hi in slabs]
    gathers = [_sc_gather2(hi - lo, H) for lo, hi in slabs]
    scatters = [_sc_scatter_add(hi - lo, H, NP) for lo, hi in slabs]
    zeros_chunk = jnp.zeros((_KZ, H), f32)

    blocks = params["blocks"]
    w0 = blocks[0]["edge"]["W0"]
    a, b = pl.pallas_call(
        _proj_kernel,
        grid=(NP // BN,),
        in_specs=[_rows(BN, H), _full((H, H)), _full((H, H))],
        out_specs=(_rows(BN, H), _rows(BN, H)),
        out_shape=(jax.ShapeDtypeStruct((NP, H), f32),
                   jax.ShapeDtypeStruct((NP, H), f32)),
    )(latents, w0[H:2 * H], w0[2 * H:])

    nodes = latents
    for kb, bp in enumerate(blocks):
        w0 = bp["edge"]["W0"]
        eb = bp["edge"]
        parts = []
        gs = [gathers[i](a, b, recv_s[i], send_s[i]) for i in range(2)]
        for i, (lo, hi) in enumerate(slabs):
            edges_s[i] = pl.pallas_call(
                _edge_mlp_kernel,
                grid=((hi - lo) // BE,),
                in_specs=[_rows(BE, H), _rows(BE, H),
                          _full((H, H)), _full((1, H)), _full((H, H)),
                          _full((1, H)), _full((H, H)), _full((1, H))],
                out_specs=_rows(BE, H),
                out_shape=jax.ShapeDtypeStruct((hi - lo, H), f32),
            )(edges_s[i], gs[i], w0[:H], eb["b0"][None], eb["W1"],
              eb["b1"][None], eb["W2"], eb["b2"][None])
            parts.append(scatters[i](edges_s[i], recv_s[i], zeros_chunk))

        nd = bp["node"]
        w0n = blocks[(kb + 1) % len(blocks)]["edge"]["W0"]
        nodes, a, b = pl.pallas_call(
            _node_kernel,
            grid=(NP // BN,),
            in_specs=[_rows(BN, H), _rows(BN, H), _rows(BN, H), _rows(BN, H),
                      _rows(BN, H), _full((H, H)), _full((H, H)),
                      _full((1, H)), _full((H, H)), _full((1, H)),
                      _full((H, H)), _full((1, H)), _full((H, H)),
                      _full((H, H))],
            out_specs=(_rows(BN, H), _rows(BN, H), _rows(BN, H)),
            out_shape=(jax.ShapeDtypeStruct((NP, H), f32),
                       jax.ShapeDtypeStruct((NP, H), f32),
                       jax.ShapeDtypeStruct((NP, H), f32)),
        )(nodes, parts[0][0], parts[0][1], parts[1][0], parts[1][1],
          nd["W0"][:H], nd["W0"][H:], nd["b0"][None], nd["W1"],
          nd["b1"][None], nd["W2"], nd["b2"][None],
          w0n[H:2 * H], w0n[2 * H:])

    dec = params["decoder"]
    w0q = jnp.zeros((8, H), f32).at[:3].set(dec["W0"][H:H + 3])
    w2d = jnp.zeros((H, 8), f32).at[:, :3].set(dec["W2"])
    b2d = jnp.zeros((1, 8), f32).at[0, :3].set(dec["b2"])
    out8 = pl.pallas_call(
        _decode_kernel,
        grid=(NX // BX,),
        in_specs=[_rows(BX, 8), _full((NP, 8)), _full((NP, H)), _full((H, H)),
                  _full((8, H)), _full((1, H)), _full((H, H)), _full((1, H)),
                  _full((H, 8)), _full((1, 8))],
        out_specs=_rows(BX, 8),
        out_shape=jax.ShapeDtypeStruct((NX, 8), f32),
    )(q8, pos8, nodes, dec["W0"][:H], w0q, dec["b0"][None], dec["W1"],
      dec["b1"][None], w2d, b2d)

    return out8[:, :3].reshape(1, NX, 3)


# speedup vs baseline: 1.0691x; 1.0691x over previous
import functools

import jax
import jax.numpy as jnp
from jax import lax
from jax.experimental import pallas as pl
from jax.experimental.pallas import tpu as pltpu
from jax.experimental.pallas import tpu_sc as plsc

H = 128
_NC = 2
_NS = 16
_NW = _NC * _NS
_K = 80


def _full(shape):
    return pl.BlockSpec(shape, lambda i: (0,) * len(shape))


def _rows(bsize, ncols):
    return pl.BlockSpec((bsize, ncols), lambda i: (i, 0))


def _softmax_scores(pts, pos):
    cross = lax.dot_general(pts, pos, (((1,), (1,)), ((), ())))
    pn = jnp.sum(pos * pos, axis=1)[None, :]
    logits = 2.0 * cross - pn
    m = jnp.max(logits, axis=1, keepdims=True)
    e = jnp.exp(logits - m)
    return e / jnp.sum(e, axis=1, keepdims=True)


def _enc_latents_kernel(xs_ref, pos_ref, w0_ref, b0_ref, w1_ref, b1_ref,
                        w2_ref, b2_ref, out_ref):
    i = pl.program_id(0)
    xs = xs_ref[...]
    h = jnp.maximum(jnp.dot(xs, w0_ref[...]) + b0_ref[...], 0.0)
    h = jnp.maximum(jnp.dot(h, w1_ref[...]) + b1_ref[...], 0.0)
    emb = jnp.dot(h, w2_ref[...]) + b2_ref[...]
    scores = _softmax_scores(xs, pos_ref[...])
    contrib = lax.dot_general(scores, emb, (((0,), (0,)), ((), ())))

    @pl.when(i == 0)
    def _():
        out_ref[...] = jnp.zeros_like(out_ref)

    out_ref[...] += contrib


def _proj_kernel(n_ref, wr_ref, ws_ref, a_ref, b_ref):
    n = n_ref[...]
    a_ref[...] = jnp.dot(n, wr_ref[...]).astype(a_ref.dtype)
    b_ref[...] = jnp.dot(n, ws_ref[...]).astype(b_ref.dtype)


def _edge_mlp_kernel(e_ref, g_ref, w0_ref, b0_ref, w1_ref, b1_ref,
                     w2_ref, b2_ref, out_ref):
    h = jnp.dot(e_ref[...], w0_ref[...]) + g_ref[...] + b0_ref[...]
    h = jnp.maximum(h, 0.0)
    h = jnp.maximum(jnp.dot(h, w1_ref[...]) + b1_ref[...], 0.0)
    out_ref[...] = jnp.dot(h, w2_ref[...]) + b2_ref[...]


def _edge_enc_mlp_kernel(e0_ref, g_ref, ew0_ref, eb0_ref, ew1_ref, eb1_ref,
                         ew2_ref, eb2_ref, w0_ref, b0_ref, w1_ref, b1_ref,
                         w2_ref, b2_ref, out_ref):
    e = jnp.maximum(e0_ref[...] * ew0_ref[...] + eb0_ref[...], 0.0)
    e = jnp.maximum(jnp.dot(e, ew1_ref[...]) + eb1_ref[...], 0.0)
    e = jnp.dot(e, ew2_ref[...]) + eb2_ref[...]
    h = jnp.maximum(jnp.dot(e, w0_ref[...]) + g_ref[...] + b0_ref[...], 0.0)
    h = jnp.maximum(jnp.dot(h, w1_ref[...]) + b1_ref[...], 0.0)
    out_ref[...] = jnp.dot(h, w2_ref[...]) + b2_ref[...]


def _node_kernel(n_ref, p0_ref, p1_ref, p2_ref, p3_ref, w0a_ref, w0b_ref,
                 b0_ref, w1_ref, b1_ref, w2_ref, b2_ref, wr_ref, ws_ref,
                 out_ref, a_ref, b_ref):
    n = n_ref[...]
    inbox = (p0_ref[...] + p1_ref[...]) + (p2_ref[...] + p3_ref[...])
    h = jnp.maximum(jnp.dot(n, w0a_ref[...]) + jnp.dot(inbox, w0b_ref[...])
                    + b0_ref[...], 0.0)
    h = jnp.maximum(jnp.dot(h, w1_ref[...]) + b1_ref[...], 0.0)
    nn = n + jnp.dot(h, w2_ref[...]) + b2_ref[...]
    out_ref[...] = nn
    a_ref[...] = jnp.dot(nn, wr_ref[...])
    b_ref[...] = jnp.dot(nn, ws_ref[...])


def _decode_kernel(qs_ref, pos_ref, lat_ref, w0z_ref, w0q_ref, b0_ref,
                   w1_ref, b1_ref, w2_ref, b2_ref, out_ref):
    qs = qs_ref[...]
    scores = _softmax_scores(qs, pos_ref[...])
    z = jnp.dot(scores, lat_ref[...])
    h = jnp.maximum(jnp.dot(z, w0z_ref[...]) + jnp.dot(qs, w0q_ref[...])
                    + b0_ref[...], 0.0)
    h = jnp.maximum(jnp.dot(h, w1_ref[...]) + b1_ref[...], 0.0)
    out_ref[...] = jnp.dot(h, w2_ref[...]) + b2_ref[...]


def _pick_k(per_w):
    for k in range(128, 7, -8):
        if per_w % k == 0:
            return k
    return 8


def _ring(per_w):
    k = _pick_k(per_w)
    n_chunks = per_w // k
    nb = min(6, n_chunks)
    return k, n_chunks, nb


def _run_ring(n_chunks, nb, fire0, steps):
    n_groups = n_chunks // nb - 1
    for b in range(nb):
        fire0(b, b)

    def group(j0, carry):
        for b in range(nb):
            c = nb * j0 + b
            steps(c, b)
            fire0(c + nb, b)
        return carry

    lax.fori_loop(0, n_groups, group, 0)
    for c in range(nb * n_groups, n_chunks):
        b = c % nb
        steps(c, b)
        nxt = c + nb
        if nxt < n_chunks:
            fire0(nxt, b)


def _sc_edge_len(E, NP):
    per_w = E // _NW
    mesh = plsc.VectorSubcoreMesh(core_axis_name="c", subcore_axis_name="s",
                                  num_cores=_NC, num_subcores=_NS)

    @functools.partial(
        pl.kernel,
        out_type=jax.ShapeDtypeStruct((E,), jnp.float32),
        mesh=mesh,
        compiler_params=pltpu.CompilerParams(needs_layout_passes=False),
        scratch_types=[
            pltpu.VMEM((NP,), jnp.float32),
            pltpu.VMEM((NP,), jnp.float32),
            pltpu.VMEM((NP,), jnp.float32),
            pltpu.VMEM((per_w,), jnp.int32),
            pltpu.VMEM((per_w,), jnp.int32),
            pltpu.VMEM((per_w,), jnp.float32),
        ],
    )
    def elen(px_hbm, py_hbm, pz_hbm, ir_hbm, is_hbm, out_hbm,
             px_v, py_v, pz_v, ir_v, is_v, e_v):
        wid = lax.axis_index("s") * _NC + lax.axis_index("c")
        base = wid * per_w
        pltpu.sync_copy(px_hbm, px_v)
        pltpu.sync_copy(py_hbm, py_v)
        pltpu.sync_copy(pz_hbm, pz_v)
        pltpu.sync_copy(ir_hbm.at[pl.ds(base, per_w)], ir_v)
        pltpu.sync_copy(is_hbm.at[pl.ds(base, per_w)], is_v)

        def body(t, carry):
            ir = ir_v[pl.ds(t * 16, 16)]
            js = is_v[pl.ds(t * 16, 16)]
            dx = plsc.load_gather(px_v, [ir]) - plsc.load_gather(px_v, [js])
            dy = plsc.load_gather(py_v, [ir]) - plsc.load_gather(py_v, [js])
            dz = plsc.load_gather(pz_v, [ir]) - plsc.load_gather(pz_v, [js])
            e_v[pl.ds(t * 16, 16)] = dx * dx + dy * dy + dz * dz
            return carry

        lax.fori_loop(0, per_w // 16, body, 0)
        pltpu.sync_copy(e_v, out_hbm.at[pl.ds(base, per_w)])

    return elen


def _sc_gather2(E, D):
    per_w = E // _NW
    K, n_chunks, NB = _ring(per_w)
    n_groups = n_chunks // NB - 1
    f32 = jnp.float32
    mesh = plsc.VectorSubcoreMesh(core_axis_name="c", subcore_axis_name="s",
                                  num_cores=_NC, num_subcores=_NS)

    @functools.partial(
        pl.kernel,
        out_type=jax.ShapeDtypeStruct((E, D), f32),
        mesh=mesh,
        scratch_types=(
            [pltpu.VMEM((per_w,), jnp.int32)] * 2
            + [pltpu.VMEM((K, D), f32)] * NB
            + [pltpu.SemaphoreType.DMA] * (3 * NB)
        ),
    )
    def gather2(t1_hbm, t2_hbm, i1_hbm, i2_hbm, o_hbm, *scr):
        i1_v, i2_v = scr[0], scr[1]
        r1 = scr[2:2 + NB]
        sg = scr[2 + NB:2 + 2 * NB]
        sa = scr[2 + 2 * NB:2 + 3 * NB]
        sw = scr[2 + 3 * NB:2 + 4 * NB]
        wid = lax.axis_index("s") * _NC + lax.axis_index("c")
        base = wid * per_w
        pltpu.sync_copy(i1_hbm.at[pl.ds(base, per_w)], i1_v)
        pltpu.sync_copy(i2_hbm.at[pl.ds(base, per_w)], i2_v)

        def fire_g(c, b):
            pltpu.async_copy(t1_hbm.at[i1_v.at[pl.ds(c * K, K)]], r1[b], sg[b])

        def wait_g(b):
            pltpu.make_async_copy(t1_hbm.at[pl.ds(0, K)], r1[b], sg[b]).wait()

        def fire_a(c, b):
            pltpu.async_copy(t2_hbm.at[i2_v.at[pl.ds(c * K, K)]], r1[b],
                             sa[b], add=True)

        def wait_a(b):
            pltpu.make_async_copy(t2_hbm.at[pl.ds(0, K)], r1[b], sa[b]).wait()

        def fire_w(c, b):
            pltpu.async_copy(r1[b], o_hbm.at[pl.ds(base + c * K, K)], sw[b])

        def wait_w(c, b):
            pltpu.make_async_copy(r1[b], o_hbm.at[pl.ds(base + c * K, K)],
                                  sw[b]).wait()

        def steps(c, b):
            wait_g(b)
            fire_a(c, b)
            wait_a(b)
            fire_w(c, b)
            wait_w(c, b)

        _run_ring(n_chunks, NB, fire_g, steps)

    return gather2


_KZ = 80


def _sc_scatter_add(E, D, n_rows):
    per_w = E // _NW
    K = min(_pick_k(per_w), 80)
    n_chunks = per_w // K
    NB = min(4, n_chunks)
    rows_per_tile = n_rows // _NS
    zchunks = rows_per_tile // _KZ
    f32 = jnp.float32
    mesh = plsc.VectorSubcoreMesh(core_axis_name="c", subcore_axis_name="s",
                                  num_cores=_NC, num_subcores=_NS)

    @functools.partial(
        pl.kernel,
        out_type=jax.ShapeDtypeStruct((_NC, n_rows, D), f32),
        mesh=mesh,
        scratch_types=(
            [pltpu.VMEM((K,), jnp.int32)] * NB
            + [pltpu.VMEM((K, D), f32)] * NB
            + [pltpu.SemaphoreType.DMA] * (2 * NB)
            + [pltpu.VMEM_SHARED((n_rows, D), f32)]
        ),
    )
    def scatter(vals_hbm, idx_hbm, zeros_hbm, out_hbm, *scr):
        ib = scr[0:NB]
        r = scr[NB:2 * NB]
        sl = scr[2 * NB:3 * NB]
        sa = scr[3 * NB:4 * NB]
        acc_sh = scr[4 * NB]
        cid = lax.axis_index("c")
        sid = lax.axis_index("s")
        wid = sid * _NC + cid
        base = wid * per_w
        row0 = sid * rows_per_tile

        pltpu.sync_copy(zeros_hbm, r[0].at[pl.ds(0, _KZ)])

        def zbody(j, carry):
            pltpu.sync_copy(r[0].at[pl.ds(0, _KZ)],
                            acc_sh.at[pl.ds(row0 + j * _KZ, _KZ)])
            return carry

        def fire_l(c, b):
            off = base + c * K
            pltpu.async_copy(idx_hbm.at[pl.ds(off, K)], ib[b], sl[b])
            pltpu.async_copy(vals_hbm.at[pl.ds(off, K)], r[b], sl[b])

        def wait_l(b):
            pltpu.make_async_copy(idx_hbm.at[pl.ds(0, K)], ib[b], sl[b]).wait()
            pltpu.make_async_copy(vals_hbm.at[pl.ds(base, K)], r[b],
                                  sl[b]).wait()

        def fire_s(c, b):
            pltpu.async_copy(r[b], acc_sh.at[ib[b]], sa[b], add=True)

        def wait_s(c, b):
            pltpu.make_async_copy(r[b], acc_sh.at[ib[b]], sa[b]).wait()

        lax.fori_loop(0, zchunks, zbody, 0)
        plsc.subcore_barrier()

        def steps(c, b):
            wait_l(b)
            fire_s(c, b)
            wait_s(c, b)

        _run_ring(n_chunks, NB, fire_l, steps)
        plsc.subcore_barrier()

        def dbody(j, carry):
            r0 = row0 + j * _KZ
            pltpu.sync_copy(acc_sh.at[pl.ds(r0, _KZ)],
                            r[0].at[pl.ds(0, _KZ)])
            pltpu.sync_copy(r[0].at[pl.ds(0, _KZ)],
                            out_hbm.at[cid, pl.ds(r0, _KZ)])
            return carry

        lax.fori_loop(0, zchunks, dbody, 0)

    return scatter


def kernel(x, s, q, pos, senders, receivers, params):
    f32 = jnp.float32
    N = pos.shape[0]
    E = senders.shape[0]
    NX = x.shape[1]
    NP = ((N + _NS * _K - 1) // (_NS * _K)) * (_NS * _K)
    BX = 256
    BE = 2560
    BN = 2560

    senders = senders.astype(jnp.int32)
    receivers = receivers.astype(jnp.int32)

    pos8 = jnp.zeros((NP, 8), f32).at[:N, :3].set(pos).at[N:, 0].set(1e4)
    xs8 = jnp.zeros((NX, 8), f32).at[:, :3].set(x[0]).at[:, 3:6].set(s[0])
    q8 = jnp.zeros((NX, 8), f32).at[:, :3].set(q[0])
    posx = jnp.zeros((NP,), f32).at[:N].set(pos[:, 0])
    posy = jnp.zeros((NP,), f32).at[:N].set(pos[:, 1])
    posz = jnp.zeros((NP,), f32).at[:N].set(pos[:, 2])

    enc = params["encoder"]
    w0e = jnp.zeros((8, H), f32).at[:6].set(enc["W0"])
    latents = pl.pallas_call(
        _enc_latents_kernel,
        grid=(NX // BX,),
        in_specs=[_rows(BX, 8), _full((NP, 8)), _full((8, H)), _full((1, H)),
                  _full((H, H)), _full((1, H)), _full((H, H)), _full((1, H))],
        out_specs=_full((NP, H)),
        out_shape=jax.ShapeDtypeStruct((NP, H), f32),
    )(xs8, pos8, w0e, enc["b0"][None], enc["W1"], enc["b1"][None],
      enc["W2"], enc["b2"][None])

    e0 = _sc_edge_len(E, NP)(posx, posy, posz, receivers, senders)

    q = _NW * _K
    e_a = ((E // q + 1) // 2) * q
    slabs = [(0, e_a), (e_a, E)]

    ee = params["edge_enc"]
    e0_s = [e0[lo:hi, None] for lo, hi in slabs]
    edges_s = [None, None]

    recv_s = [receivers[lo:hi] for lo, hi in slabs]
    send_s = [senders[lo:hi] for lo, hi in slabs]
    gathers = [_sc_gather2(hi - lo, H) for lo, hi in slabs]
    scatters = [_sc_scatter_add(hi - lo, H, NP) for lo, hi in slabs]
    zeros_chunk = jnp.zeros((_KZ, H), f32)

    blocks = params["blocks"]
    w0 = blocks[0]["edge"]["W0"]
    a, b = pl.pallas_call(
        _proj_kernel,
        grid=(NP // BN,),
        in_specs=[_rows(BN, H), _full((H, H)), _full((H, H))],
        out_specs=(_rows(BN, H), _rows(BN, H)),
        out_shape=(jax.ShapeDtypeStruct((NP, H), f32),
                   jax.ShapeDtypeStruct((NP, H), f32)),
    )(latents, w0[H:2 * H], w0[2 * H:])

    nodes = latents
    for kb, bp in enumerate(blocks):
        w0 = bp["edge"]["W0"]
        eb = bp["edge"]
        parts = []
        gs = [gathers[i](a, b, recv_s[i], send_s[i]) for i in range(2)]
        for i, (lo, hi) in enumerate(slabs):
            blk_w = [w0[:H], eb["b0"][None], eb["W1"], eb["b1"][None],
                     eb["W2"], eb["b2"][None]]
            blk_specs = [_full((H, H)), _full((1, H)), _full((H, H)),
                         _full((1, H)), _full((H, H)), _full((1, H))]
            if kb == 0:
                edges_s[i] = pl.pallas_call(
                    _edge_enc_mlp_kernel,
                    grid=((hi - lo) // BE,),
                    in_specs=([_rows(BE, 1), _rows(BE, H), _full((1, H)),
                               _full((1, H)), _full((H, H)), _full((1, H)),
                               _full((H, H)), _full((1, H))] + blk_specs),
                    out_specs=_rows(BE, H),
                    out_shape=jax.ShapeDtypeStruct((hi - lo, H), f32),
                )(e0_s[i], gs[i], ee["W0"], ee["b0"][None], ee["W1"],
                  ee["b1"][None], ee["W2"], ee["b2"][None], *blk_w)
            else:
                edges_s[i] = pl.pallas_call(
                    _edge_mlp_kernel,
                    grid=((hi - lo) // BE,),
                    in_specs=[_rows(BE, H), _rows(BE, H)] + blk_specs,
                    out_specs=_rows(BE, H),
                    out_shape=jax.ShapeDtypeStruct((hi - lo, H), f32),
                )(edges_s[i], gs[i], *blk_w)
            parts.append(scatters[i](edges_s[i], recv_s[i], zeros_chunk))

        nd = bp["node"]
        w0n = blocks[(kb + 1) % len(blocks)]["edge"]["W0"]
        nodes, a, b = pl.pallas_call(
            _node_kernel,
            grid=(NP // BN,),
            in_specs=[_rows(BN, H), _rows(BN, H), _rows(BN, H), _rows(BN, H),
                      _rows(BN, H), _full((H, H)), _full((H, H)),
                      _full((1, H)), _full((H, H)), _full((1, H)),
                      _full((H, H)), _full((1, H)), _full((H, H)),
                      _full((H, H))],
            out_specs=(_rows(BN, H), _rows(BN, H), _rows(BN, H)),
            out_shape=(jax.ShapeDtypeStruct((NP, H), f32),
                       jax.ShapeDtypeStruct((NP, H), f32),
                       jax.ShapeDtypeStruct((NP, H), f32)),
        )(nodes, parts[0][0], parts[0][1], parts[1][0], parts[1][1],
          nd["W0"][:H], nd["W0"][H:], nd["b0"][None], nd["W1"],
          nd["b1"][None], nd["W2"], nd["b2"][None],
          w0n[H:2 * H], w0n[2 * H:])

    dec = params["decoder"]
    w0q = jnp.zeros((8, H), f32).at[:3].set(dec["W0"][H:H + 3])
    w2d = jnp.zeros((H, 8), f32).at[:, :3].set(dec["W2"])
    b2d = jnp.zeros((1, 8), f32).at[0, :3].set(dec["b2"])
    out8 = pl.pallas_call(
        _decode_kernel,
        grid=(NX // BX,),
        in_specs=[_rows(BX, 8), _full((NP, 8)), _full((NP, H)), _full((H, H)),
                  _full((8, H)), _full((1, H)), _full((H, H)), _full((1, H)),
                  _full((H, 8)), _full((1, 8))],
        out_specs=_rows(BX, 8),
        out_shape=jax.ShapeDtypeStruct((NX, 8), f32),
    )(q8, pos8, nodes, dec["W0"][:H], w0q, dec["b0"][None], dec["W1"],
      dec["b1"][None], w2d, b2d)

    return out8[:, :3].reshape(1, NX, 3)
